# Initial kernel scaffold; baseline (speedup 1.0000x reference)
import jax
import jax.numpy as jnp
from jax.experimental import pallas as pl


def kernel(node_embeddings, adjacency_list):
    raise NotImplementedError("write your pallas kernel here")


# speedup vs baseline: 11.9212x; 11.9212x over previous
import functools

import jax
import jax.numpy as jnp
from jax import lax
from jax.experimental import pallas as pl
from jax.experimental.pallas import tpu as pltpu
from jax.experimental.pallas import tpu_sc as plsc

N_NODES = 10000
D_FEAT = 128
N_EDGES = 320000

NC = 2
NS = 16
NW = NC * NS

CHUNK = 128
CHUNKS_PER_TILE = 80
N_CHUNKS_PAD = CHUNKS_PER_TILE * NW
E_PAD = N_CHUNKS_PAD * CHUNK
DUMMY = N_NODES
NPAD = 10240
ROWS_PER_TILE = NPAD // NS
EMB_PAD = 10016

_mesh = plsc.VectorSubcoreMesh(core_axis_name="c", subcore_axis_name="s")


def _degree_body(src_hbm, tgt_hbm, din_hbm, dout_hbm,
                 src_v, tgt_v, ones_v, din_sp, dout_sp):
    c = lax.axis_index("c")
    s = lax.axis_index("s")
    wid = c * NS + s
    start = wid * CHUNKS_PER_TILE

    for k in range(CHUNK // 16):
        ones_v[pl.ds(k * 16, 16)] = jnp.zeros((16,), jnp.float32)
    zbase = s * ROWS_PER_TILE
    for k in range(ROWS_PER_TILE // CHUNK):
        pltpu.sync_copy(ones_v, din_sp.at[pl.ds(zbase + k * CHUNK, CHUNK)])
        pltpu.sync_copy(ones_v, dout_sp.at[pl.ds(zbase + k * CHUNK, CHUNK)])
    for k in range(CHUNK // 16):
        ones_v[pl.ds(k * 16, 16)] = jnp.ones((16,), jnp.float32)

    pltpu.sync_copy(src_hbm.at[pl.ds(start, CHUNKS_PER_TILE)], src_v)
    pltpu.sync_copy(tgt_hbm.at[pl.ds(start, CHUNKS_PER_TILE)], tgt_v)
    plsc.subcore_barrier()

    def step(t, _):
        pltpu.sync_copy(ones_v, din_sp.at[tgt_v.at[t]], add=True)
        pltpu.sync_copy(ones_v, dout_sp.at[src_v.at[t]], add=True)
        return _
    lax.fori_loop(0, CHUNKS_PER_TILE, step, None)
    plsc.subcore_barrier()

    base = s * ROWS_PER_TILE
    pltpu.sync_copy(din_sp.at[pl.ds(base, ROWS_PER_TILE)],
                    din_hbm.at[pl.ds(c * NPAD + base, ROWS_PER_TILE)])
    pltpu.sync_copy(dout_sp.at[pl.ds(base, ROWS_PER_TILE)],
                    dout_hbm.at[pl.ds(c * NPAD + base, ROWS_PER_TILE)])


_degree_kernel = pl.kernel(
    _degree_body,
    out_type=(jax.ShapeDtypeStruct((NC * NPAD,), jnp.float32),
              jax.ShapeDtypeStruct((NC * NPAD,), jnp.float32)),
    mesh=_mesh,
    scratch_types=[
        pltpu.VMEM((CHUNKS_PER_TILE, CHUNK), jnp.int32),
        pltpu.VMEM((CHUNKS_PER_TILE, CHUNK), jnp.int32),
        pltpu.VMEM((CHUNK,), jnp.float32),
        pltpu.VMEM_SHARED((NPAD,), jnp.float32),
        pltpu.VMEM_SHARED((NPAD,), jnp.float32),
    ],
)


def _prescale_body(emb_ref, doutT_ref, out_ref):
    d = doutT_ref[...]
    b = lax.rsqrt(jnp.maximum(d[:, 0:1] + d[:, 1:2], 1.0))
    out_ref[pl.ds(0, N_NODES), :] = emb_ref[...] * b[0:N_NODES, :]
    out_ref[pl.ds(N_NODES, EMB_PAD - N_NODES), :] = jnp.zeros(
        (EMB_PAD - N_NODES, D_FEAT), jnp.float32)


_prescale_kernel = pl.pallas_call(
    _prescale_body,
    out_shape=jax.ShapeDtypeStruct((EMB_PAD, D_FEAT), jnp.float32),
)


def _aggregate_body(emb2_hbm, src_hbm, tgt_hbm, p_hbm,
                    src_v, tgt_v, rows_v, out_sp, sem):
    c = lax.axis_index("c")
    s = lax.axis_index("s")
    wid = c * NS + s
    start = wid * CHUNKS_PER_TILE

    def zrow(i, carry):
        for k in range(D_FEAT // 16):
            rows_v[i, pl.ds(k * 16, 16)] = jnp.zeros((16,), jnp.float32)
        return carry
    lax.fori_loop(0, CHUNK, zrow, None)
    zbase = s * ROWS_PER_TILE
    for k in range(ROWS_PER_TILE // CHUNK):
        pltpu.sync_copy(rows_v, out_sp.at[pl.ds(zbase + k * CHUNK, CHUNK)])

    pltpu.sync_copy(src_hbm.at[pl.ds(start, CHUNKS_PER_TILE)], src_v)
    pltpu.sync_copy(tgt_hbm.at[pl.ds(start, CHUNKS_PER_TILE)], tgt_v)
    plsc.subcore_barrier()

    def step(t, _):
        pltpu.async_copy(emb2_hbm.at[src_v.at[t]], rows_v, sem).wait()
        pltpu.sync_copy(rows_v, out_sp.at[tgt_v.at[t]], add=True)
        return _
    lax.fori_loop(0, CHUNKS_PER_TILE, step, None)
    plsc.subcore_barrier()

    base = s * ROWS_PER_TILE
    pltpu.sync_copy(out_sp.at[pl.ds(base, ROWS_PER_TILE)],
                    p_hbm.at[c, pl.ds(base, ROWS_PER_TILE)])


_aggregate_kernel = pl.kernel(
    _aggregate_body,
    out_type=jax.ShapeDtypeStruct((NC, NPAD, D_FEAT), jnp.float32),
    mesh=_mesh,
    scratch_types=[
        pltpu.VMEM((CHUNKS_PER_TILE, CHUNK), jnp.int32),
        pltpu.VMEM((CHUNKS_PER_TILE, CHUNK), jnp.int32),
        pltpu.VMEM((CHUNK, D_FEAT), jnp.float32),
        pltpu.VMEM_SHARED((NPAD, D_FEAT), jnp.float32),
        pltpu.SemaphoreType.DMA,
    ],
)


def _postscale_body(p_ref, dinT_ref, out_ref):
    d = dinT_ref[...]
    a = lax.rsqrt(jnp.maximum(d[:, 0:1] + d[:, 1:2], 1.0))
    tot = p_ref[0] + p_ref[1]
    out_ref[...] = a[0:N_NODES, :] * tot[0:N_NODES, :]


_postscale_kernel = pl.pallas_call(
    _postscale_body,
    out_shape=jax.ShapeDtypeStruct((N_NODES, D_FEAT), jnp.float32),
)


def kernel(node_embeddings, adjacency_list):
    adj = adjacency_list.astype(jnp.int32)
    pad = jnp.full((E_PAD - N_EDGES,), DUMMY, jnp.int32)
    src = jnp.concatenate([adj[:, 0], pad]).reshape(N_CHUNKS_PAD, CHUNK)
    tgt = jnp.concatenate([adj[:, 1], pad]).reshape(N_CHUNKS_PAD, CHUNK)

    din_p, dout_p = _degree_kernel(src, tgt)
    emb2 = _prescale_kernel(node_embeddings, dout_p.reshape(NC, NPAD).T)
    p = _aggregate_kernel(emb2, src, tgt)
    out = _postscale_kernel(p, din_p.reshape(NC, NPAD).T)
    return out


# speedup vs baseline: 26.2979x; 2.2060x over previous
import functools

import jax
import jax.numpy as jnp
from jax import lax
from jax.experimental import pallas as pl
from jax.experimental.pallas import tpu as pltpu
from jax.experimental.pallas import tpu_sc as plsc

N_NODES = 10000
D_FEAT = 128
N_EDGES = 320000

NC = 2
NS = 16
NW = NC * NS

CHUNK = 128
CHUNKS_PER_TILE = 80
N_CHUNKS_PAD = CHUNKS_PER_TILE * NW
E_PAD = N_CHUNKS_PAD * CHUNK
DUMMY = N_NODES
NPAD = 10240
ROWS_PER_TILE = NPAD // NS
EMB_PAD = NPAD
N_DUMMY = NPAD - N_NODES

_mesh = plsc.VectorSubcoreMesh(core_axis_name="c", subcore_axis_name="s")


def _degree_body(src_hbm, tgt_hbm, din_hbm, dout_hbm,
                 src_v, tgt_v, ones_v, din_sp, dout_sp):
    c = lax.axis_index("c")
    s = lax.axis_index("s")
    wid = c * NS + s
    start = wid * CHUNKS_PER_TILE

    for k in range(CHUNK // 16):
        ones_v[pl.ds(k * 16, 16)] = jnp.zeros((16,), jnp.float32)
    zbase = s * ROWS_PER_TILE
    for k in range(ROWS_PER_TILE // CHUNK):
        pltpu.sync_copy(ones_v, din_sp.at[pl.ds(zbase + k * CHUNK, CHUNK)])
        pltpu.sync_copy(ones_v, dout_sp.at[pl.ds(zbase + k * CHUNK, CHUNK)])
    for k in range(CHUNK // 16):
        ones_v[pl.ds(k * 16, 16)] = jnp.ones((16,), jnp.float32)

    pltpu.sync_copy(src_hbm.at[pl.ds(start, CHUNKS_PER_TILE)], src_v)
    pltpu.sync_copy(tgt_hbm.at[pl.ds(start, CHUNKS_PER_TILE)], tgt_v)
    plsc.subcore_barrier()

    def step(t, _):
        pltpu.sync_copy(ones_v, din_sp.at[tgt_v.at[t]], add=True)
        pltpu.sync_copy(ones_v, dout_sp.at[src_v.at[t]], add=True)
        return _
    lax.fori_loop(0, CHUNKS_PER_TILE, step, None)
    plsc.subcore_barrier()

    base = s * ROWS_PER_TILE
    pltpu.sync_copy(din_sp.at[pl.ds(base, ROWS_PER_TILE)],
                    din_hbm.at[pl.ds(c * NPAD + base, ROWS_PER_TILE)])
    pltpu.sync_copy(dout_sp.at[pl.ds(base, ROWS_PER_TILE)],
                    dout_hbm.at[pl.ds(c * NPAD + base, ROWS_PER_TILE)])


_degree_kernel = pl.kernel(
    _degree_body,
    out_type=(jax.ShapeDtypeStruct((NC * NPAD,), jnp.float32),
              jax.ShapeDtypeStruct((NC * NPAD,), jnp.float32)),
    mesh=_mesh,
    scratch_types=[
        pltpu.VMEM((CHUNKS_PER_TILE, CHUNK), jnp.int32),
        pltpu.VMEM((CHUNKS_PER_TILE, CHUNK), jnp.int32),
        pltpu.VMEM((CHUNK,), jnp.float32),
        pltpu.VMEM_SHARED((NPAD,), jnp.float32),
        pltpu.VMEM_SHARED((NPAD,), jnp.float32),
    ],
)


def _prescale_body(emb_ref, doutT_ref, out_ref):
    d = doutT_ref[...]
    b = lax.rsqrt(jnp.maximum(d[:, 0:1] + d[:, 1:2], 1.0))
    out_ref[pl.ds(0, N_NODES), :] = emb_ref[...] * b[0:N_NODES, :]
    out_ref[pl.ds(N_NODES, EMB_PAD - N_NODES), :] = jnp.zeros(
        (EMB_PAD - N_NODES, D_FEAT), jnp.float32)


_prescale_kernel = pl.pallas_call(
    _prescale_body,
    out_shape=jax.ShapeDtypeStruct((EMB_PAD, D_FEAT), jnp.float32),
)


def _aggregate_body(emb2_hbm, src_hbm, tgt_hbm, p_hbm,
                    src_v, tgt_v, rows_v, out_sp, sem):
    c = lax.axis_index("c")
    s = lax.axis_index("s")
    wid = c * NS + s
    start = wid * CHUNKS_PER_TILE

    def zrow(i, carry):
        for k in range(D_FEAT // 16):
            rows_v[i, pl.ds(k * 16, 16)] = jnp.zeros((16,), jnp.float32)
        return carry
    lax.fori_loop(0, CHUNK, zrow, None)
    zbase = s * ROWS_PER_TILE
    for k in range(ROWS_PER_TILE // CHUNK):
        pltpu.sync_copy(rows_v, out_sp.at[pl.ds(zbase + k * CHUNK, CHUNK)])

    pltpu.sync_copy(src_hbm.at[pl.ds(start, CHUNKS_PER_TILE)], src_v)
    pltpu.sync_copy(tgt_hbm.at[pl.ds(start, CHUNKS_PER_TILE)], tgt_v)
    plsc.subcore_barrier()

    def step(t, _):
        pltpu.async_copy(emb2_hbm.at[src_v.at[t]], rows_v, sem).wait()
        pltpu.sync_copy(rows_v, out_sp.at[tgt_v.at[t]], add=True)
        return _
    lax.fori_loop(0, CHUNKS_PER_TILE, step, None)
    plsc.subcore_barrier()

    base = s * ROWS_PER_TILE
    pltpu.sync_copy(out_sp.at[pl.ds(base, ROWS_PER_TILE)],
                    p_hbm.at[c, pl.ds(base, ROWS_PER_TILE)])


_aggregate_kernel = pl.kernel(
    _aggregate_body,
    out_type=jax.ShapeDtypeStruct((NC, NPAD, D_FEAT), jnp.float32),
    mesh=_mesh,
    scratch_types=[
        pltpu.VMEM((CHUNKS_PER_TILE, CHUNK), jnp.int32),
        pltpu.VMEM((CHUNKS_PER_TILE, CHUNK), jnp.int32),
        pltpu.VMEM((CHUNK, D_FEAT), jnp.float32),
        pltpu.VMEM_SHARED((NPAD, D_FEAT), jnp.float32),
        pltpu.SemaphoreType.DMA,
    ],
)


def _postscale_body(p_ref, dinT_ref, out_ref):
    d = dinT_ref[...]
    a = lax.rsqrt(jnp.maximum(d[:, 0:1] + d[:, 1:2], 1.0))
    tot = p_ref[0] + p_ref[1]
    out_ref[...] = a[0:N_NODES, :] * tot[0:N_NODES, :]


_postscale_kernel = pl.pallas_call(
    _postscale_body,
    out_shape=jax.ShapeDtypeStruct((N_NODES, D_FEAT), jnp.float32),
)


def kernel(node_embeddings, adjacency_list):
    adj = adjacency_list.astype(jnp.int32)
    pad = DUMMY + (jnp.arange(E_PAD - N_EDGES, dtype=jnp.int32) % N_DUMMY)
    src = jnp.concatenate([adj[:, 0], pad]).reshape(N_CHUNKS_PAD, CHUNK)
    tgt = jnp.concatenate([adj[:, 1], pad]).reshape(N_CHUNKS_PAD, CHUNK)

    din_p, dout_p = _degree_kernel(src, tgt)
    emb2 = _prescale_kernel(node_embeddings, dout_p.reshape(NC, NPAD).T)
    p = _aggregate_kernel(emb2, src, tgt)
    out = _postscale_kernel(p, din_p.reshape(NC, NPAD).T)
    return out


# speedup vs baseline: 36.2847x; 1.3798x over previous
import jax
import jax.numpy as jnp
from jax import lax
from jax.experimental import pallas as pl
from jax.experimental.pallas import tpu as pltpu
from jax.experimental.pallas import tpu_sc as plsc

N_NODES = 10000
D_FEAT = 128
N_EDGES = 320000

NC = 2
NS = 16
NW = NC * NS

CHUNK = 128
N_CHUNKS = 2560
E_PAD = N_CHUNKS * CHUNK
CPT = N_CHUNKS // NW
DUMMY = N_NODES
NPAD = 10240
ROWS_PER_TILE = NPAD // NS
N_DUMMY = NPAD - N_NODES

PACK_SHIFT = 14
PACK_MASK = (1 << PACK_SHIFT) - 1

NB = 2

_mesh = plsc.VectorSubcoreMesh(core_axis_name="c", subcore_axis_name="s")


def _unpack_chunk(packed_v, t, si_row, ti_row):
    for k in range(CHUNK // 16):
        v = packed_v[t, pl.ds(k * 16, 16)]
        si_row[pl.ds(k * 16, 16)] = jax.lax.shift_right_logical(
            v, jnp.full((16,), PACK_SHIFT, jnp.int32))
        ti_row[pl.ds(k * 16, 16)] = jax.lax.bitwise_and(
            v, jnp.full((16,), PACK_MASK, jnp.int32))


def _degree_body(packed_hbm, din_hbm, dout_hbm,
                 packed_v, si_v, ti_v, ones_v, din_sp, dout_sp):
    c = lax.axis_index("c")
    s = lax.axis_index("s")
    wid = c * NS + s
    start = wid * CPT

    for k in range(CHUNK // 16):
        ones_v[pl.ds(k * 16, 16)] = jnp.zeros((16,), jnp.float32)
    zbase = s * ROWS_PER_TILE
    for k in range(ROWS_PER_TILE // CHUNK):
        pltpu.sync_copy(ones_v, din_sp.at[pl.ds(zbase + k * CHUNK, CHUNK)])
        pltpu.sync_copy(ones_v, dout_sp.at[pl.ds(zbase + k * CHUNK, CHUNK)])
    for k in range(CHUNK // 16):
        ones_v[pl.ds(k * 16, 16)] = jnp.ones((16,), jnp.float32)

    pltpu.sync_copy(packed_hbm.at[pl.ds(start, CPT)], packed_v)
    plsc.subcore_barrier()

    def step(t, carry):
        _unpack_chunk(packed_v, t, si_v, ti_v)
        pltpu.sync_copy(ones_v, din_sp.at[ti_v], add=True)
        pltpu.sync_copy(ones_v, dout_sp.at[si_v], add=True)
        return carry
    lax.fori_loop(0, CPT, step, None)
    plsc.subcore_barrier()

    base = s * ROWS_PER_TILE
    pltpu.sync_copy(din_sp.at[pl.ds(base, ROWS_PER_TILE)],
                    din_hbm.at[pl.ds(c * NPAD + base, ROWS_PER_TILE)])
    pltpu.sync_copy(dout_sp.at[pl.ds(base, ROWS_PER_TILE)],
                    dout_hbm.at[pl.ds(c * NPAD + base, ROWS_PER_TILE)])


_degree_kernel = pl.kernel(
    _degree_body,
    out_type=(jax.ShapeDtypeStruct((NC * NPAD,), jnp.float32),
              jax.ShapeDtypeStruct((NC * NPAD,), jnp.float32)),
    mesh=_mesh,
    scratch_types=[
        pltpu.VMEM((CPT, CHUNK), jnp.int32),
        pltpu.VMEM((CHUNK,), jnp.int32),
        pltpu.VMEM((CHUNK,), jnp.int32),
        pltpu.VMEM((CHUNK,), jnp.float32),
        pltpu.VMEM_SHARED((NPAD,), jnp.float32),
        pltpu.VMEM_SHARED((NPAD,), jnp.float32),
    ],
)


def _prescale_body(emb_ref, doutT_ref, out_ref):
    d = doutT_ref[...]
    b = lax.rsqrt(jnp.maximum(d[:, 0:1] + d[:, 1:2], 1.0))
    out_ref[pl.ds(0, N_NODES), :] = emb_ref[...] * b[0:N_NODES, :]
    out_ref[pl.ds(N_NODES, NPAD - N_NODES), :] = jnp.zeros(
        (NPAD - N_NODES, D_FEAT), jnp.float32)


_prescale_kernel = pl.pallas_call(
    _prescale_body,
    out_shape=jax.ShapeDtypeStruct((NPAD, D_FEAT), jnp.float32),
)


def _aggregate_body(emb2_hbm, packed_hbm, p_hbm,
                    packed_v, si_v, ti_v, rows_v, out_sp, gsem):
    c = lax.axis_index("c")
    s = lax.axis_index("s")
    wid = c * NS + s
    start = wid * CPT

    def zrow(i, carry):
        for k in range(D_FEAT // 16):
            rows_v[0, i, pl.ds(k * 16, 16)] = jnp.zeros((16,), jnp.float32)
        return carry
    lax.fori_loop(0, CHUNK, zrow, None)
    zbase = s * ROWS_PER_TILE
    for k in range(ROWS_PER_TILE // CHUNK):
        pltpu.sync_copy(rows_v.at[0], out_sp.at[pl.ds(zbase + k * CHUNK, CHUNK)])

    pltpu.sync_copy(packed_hbm.at[pl.ds(start, CPT)], packed_v)
    plsc.subcore_barrier()

    _unpack_chunk(packed_v, 0, si_v.at[0], ti_v.at[0])
    pltpu.async_copy(emb2_hbm.at[si_v.at[0]], rows_v.at[0], gsem.at[0])

    def step(g, carry):
        for u in range(NB):
            t = g * NB + u
            u2 = (u + 1) % NB
            @pl.when(t + 1 < CPT)
            def _():
                _unpack_chunk(packed_v, t + 1, si_v.at[u2], ti_v.at[u2])
                pltpu.async_copy(emb2_hbm.at[si_v.at[u2]], rows_v.at[u2],
                                 gsem.at[u2])
            pltpu.make_async_copy(emb2_hbm.at[si_v.at[u]], rows_v.at[u],
                                  gsem.at[u]).wait()
            pltpu.sync_copy(rows_v.at[u], out_sp.at[ti_v.at[u]], add=True)
        return carry
    lax.fori_loop(0, CPT // NB, step, None)
    plsc.subcore_barrier()

    base = s * ROWS_PER_TILE
    pltpu.sync_copy(out_sp.at[pl.ds(base, ROWS_PER_TILE)],
                    p_hbm.at[c, pl.ds(base, ROWS_PER_TILE)])


_aggregate_kernel = pl.kernel(
    _aggregate_body,
    out_type=jax.ShapeDtypeStruct((NC, NPAD, D_FEAT), jnp.float32),
    mesh=_mesh,
    scratch_types=[
        pltpu.VMEM((CPT, CHUNK), jnp.int32),
        pltpu.VMEM((NB, CHUNK), jnp.int32),
        pltpu.VMEM((NB, CHUNK), jnp.int32),
        pltpu.VMEM((NB, CHUNK, D_FEAT), jnp.float32),
        pltpu.VMEM_SHARED((NPAD, D_FEAT), jnp.float32),
        pltpu.SemaphoreType.DMA((NB,)),
    ],
)


def _postscale_body(p_ref, dinT_ref, out_ref):
    d = dinT_ref[...]
    a = lax.rsqrt(jnp.maximum(d[:, 0:1] + d[:, 1:2], 1.0))[0:N_NODES]
    tot = p_ref[0] + p_ref[1]
    out_ref[...] = a * tot[0:N_NODES, :]


_postscale_kernel = pl.pallas_call(
    _postscale_body,
    out_shape=jax.ShapeDtypeStruct((N_NODES, D_FEAT), jnp.float32),
)


def kernel(node_embeddings, adjacency_list):
    adj = adjacency_list.astype(jnp.int32)
    pad = DUMMY + (jnp.arange(E_PAD - N_EDGES, dtype=jnp.int32) % N_DUMMY)
    src = jnp.concatenate([adj[:, 0], pad])
    tgt = jnp.concatenate([adj[:, 1], pad])
    packed = ((src << PACK_SHIFT) | tgt).reshape(N_CHUNKS, CHUNK)

    din_p, dout_p = _degree_kernel(packed)
    emb2 = _prescale_kernel(node_embeddings, dout_p.reshape(NC, NPAD).T)
    p = _aggregate_kernel(emb2, packed)
    out = _postscale_kernel(p, din_p.reshape(NC, NPAD).T)
    return out
